# baseline (speedup 1.0000x reference)
import jax
import jax.numpy as jnp
from jax.experimental import pallas as pl
from jax.experimental.pallas import tpu as pltpu

_NUM_ROW_BLOCKS = 4


def _copy_overwrite_body(pos_ref, key_ref, value_ref, kc_ref, vc_ref,
                         ko_ref, vo_ref):
    rb = pl.program_id(1)
    l_blk = kc_ref.shape[1]
    row0 = rb * l_blk
    p = pos_ref[0] - row0
    num = key_ref.shape[1]

    ko_ref[...] = kc_ref[...]
    vo_ref[...] = vc_ref[...]

    @pl.when(jnp.logical_and(p >= 0, p + num <= l_blk))
    def _():
        ko_ref[0, pl.ds(p, num), :] = key_ref[0]
        vo_ref[0, pl.ds(p, num), :] = value_ref[0]

    @pl.when(jnp.logical_and(jnp.logical_and(p + num > 0, p < l_blk),
                             jnp.logical_not(
                                 jnp.logical_and(p >= 0, p + num <= l_blk))))
    def _():
        def body(j, _):
            r = p + j

            @pl.when(jnp.logical_and(r >= 0, r < l_blk))
            def _():
                ko_ref[0, pl.ds(r, 1), :] = key_ref[0, pl.ds(j, 1), :]
                vo_ref[0, pl.ds(r, 1), :] = value_ref[0, pl.ds(j, 1), :]
            return 0

        jax.lax.fori_loop(0, num, body, 0)


def kernel(key, value, k_cache, v_cache, next_token_pos):
    B, G, L, H = k_cache.shape
    num = key.shape[2]
    BG = B * G
    l_blk = L // _NUM_ROW_BLOCKS

    key2 = key.reshape(BG, num, H)
    value2 = value.reshape(BG, num, H)
    kc2 = k_cache.reshape(BG, L, H)
    vc2 = v_cache.reshape(BG, L, H)
    pos = jnp.asarray(next_token_pos, jnp.int32).reshape(1)

    grid = (BG, _NUM_ROW_BLOCKS)
    cache_spec = pl.BlockSpec((1, l_blk, H), lambda bg, rb: (bg, rb, 0))
    new_spec = pl.BlockSpec((1, num, H), lambda bg, rb: (bg, 0, 0))

    ko, vo = pl.pallas_call(
        _copy_overwrite_body,
        grid=grid,
        in_specs=[
            pl.BlockSpec(memory_space=pltpu.SMEM),
            new_spec,
            new_spec,
            cache_spec,
            cache_spec,
        ],
        out_specs=[cache_spec, cache_spec],
        out_shape=[
            jax.ShapeDtypeStruct((BG, L, H), k_cache.dtype),
            jax.ShapeDtypeStruct((BG, L, H), v_cache.dtype),
        ],
    )(pos, key2, value2, kc2, vc2)

    return ko.reshape(B, G, L, H), vo.reshape(B, G, L, H)
